# Initial kernel scaffold; baseline (speedup 1.0000x reference)
import jax
import jax.numpy as jnp
from jax.experimental import pallas as pl


def kernel(x, edge_index, edge_attr, params):
    raise NotImplementedError("write your pallas kernel here")


# speedup vs baseline: 4.4952x; 4.4952x over previous
import functools

import jax
import jax.numpy as jnp
from jax import lax
from jax.experimental import pallas as pl
from jax.experimental.pallas import tpu as pltpu
from jax.experimental.pallas import tpu_sc as plsc

N = 10000
E = 320000
D = 128
NC, NS = 2, 16
NW = NC * NS
CH = 128
N_CHUNKS = E // CH
RPT = 624
REM_BASE = NS * RPT
REM = N - REM_BASE


def _edge_body(x_hbm, src_hbm, dst_hbm, a_hbm, web_hbm, out_hbm,
               src_v, dst_v, a_v, rows_v, web_v, agg_sh, sem):
    c = lax.axis_index("c")
    s = lax.axis_index("s")
    w = s * NC + c

    def _zrow(i, _):
        for j in range(8):
            rows_v[i, pl.ds(j * 16, 16)] = jnp.zeros((16,), jnp.float32)
        return 0
    lax.fori_loop(0, CH, _zrow, 0)
    base_r = s * RPT
    for kk in range(RPT // CH):
        pltpu.sync_copy(rows_v, agg_sh.at[pl.ds(base_r + kk * CH, CH)])
    rem = RPT % CH
    pltpu.sync_copy(rows_v.at[pl.ds(0, rem)],
                    agg_sh.at[pl.ds(base_r + (RPT // CH) * CH, rem)])

    @pl.when(s == NS - 1)
    def _zero_tail():
        pltpu.sync_copy(rows_v.at[pl.ds(0, REM)],
                        agg_sh.at[pl.ds(REM_BASE, REM)])

    pltpu.sync_copy(web_hbm, web_v)
    plsc.subcore_barrier()

    wej = [web_v[0, pl.ds(j * 16, 16)] for j in range(8)]
    bej = [web_v[1, pl.ds(j * 16, 16)] for j in range(8)]

    nk = jnp.where(w < N_CHUNKS % NW, N_CHUNKS // NW + 1, N_CHUNKS // NW)

    def _chunk(k, _):
        eb = (w + k * NW) * CH
        pltpu.sync_copy(src_hbm.at[pl.ds(eb, CH)], src_v)
        pltpu.sync_copy(dst_hbm.at[pl.ds(eb, CH)], dst_v)
        pltpu.sync_copy(a_hbm.at[pl.ds(eb, CH)], a_v)
        pltpu.async_copy(x_hbm.at[src_v], rows_v, sem).wait()

        def _edges(i16, _):
            a16 = a_v[pl.ds(i16 * 16, 16)]
            for u in range(16):
                i = i16 * 16 + u
                a = a16[u]
                for j in range(8):
                    sl = pl.ds(j * 16, 16)
                    rows_v[i, sl] = jnp.maximum(
                        rows_v[i, sl] + (a * wej[j] + bej[j]), 0.0)
            return 0
        lax.fori_loop(0, CH // 16, _edges, 0)

        pltpu.sync_copy(rows_v, agg_sh.at[dst_v], add=True)
        return 0
    lax.fori_loop(0, nk, _chunk, 0)

    plsc.subcore_barrier()
    pltpu.sync_copy(agg_sh.at[pl.ds(s * RPT, RPT)],
                    out_hbm.at[c, pl.ds(s * RPT, RPT)])

    @pl.when(s == NS - 1)
    def _dump_tail():
        pltpu.sync_copy(agg_sh.at[pl.ds(REM_BASE, REM)],
                        out_hbm.at[c, pl.ds(REM_BASE, REM)])


@functools.cache
def _edge_call():
    return pl.kernel(
        _edge_body,
        out_type=jax.ShapeDtypeStruct((NC, N, D), jnp.float32),
        mesh=plsc.VectorSubcoreMesh(core_axis_name="c", subcore_axis_name="s",
                                    num_cores=NC, num_subcores=NS),
        scratch_types=[
            pltpu.VMEM((CH,), jnp.int32),
            pltpu.VMEM((CH,), jnp.int32),
            pltpu.VMEM((CH,), jnp.float32),
            pltpu.VMEM((CH, D), jnp.float32),
            pltpu.VMEM((2, D), jnp.float32),
            pltpu.VMEM_SHARED((N, D), jnp.float32),
            pltpu.SemaphoreType.DMA,
        ],
    )


NB = 10
RB = N // NB


def _dense1_body(eps_ref, x_ref, agg_ref, w1_ref, b1_ref,
                 h1_ref, sum_ref, sq_ref):
    x = x_ref[...]
    h = (1.0 + eps_ref[0]) * x + agg_ref[0] + agg_ref[1]
    h1 = jnp.dot(h, w1_ref[...], preferred_element_type=jnp.float32) \
        + b1_ref[...]
    h1_ref[...] = h1
    sum_ref[0] = jnp.sum(h1, axis=0, keepdims=True)
    sq_ref[0] = jnp.sum(h1 * h1, axis=0, keepdims=True)


def _dense2_body(first, x_ref, h1_ref, sum_ref, sq_ref, g_ref, bt_ref,
                 w2_ref, b2_ref, o_ref):
    h1 = h1_ref[...]
    mean = jnp.sum(sum_ref[...], axis=0) * (1.0 / N)
    ex2 = jnp.sum(sq_ref[...], axis=0) * (1.0 / N)
    var = ex2 - mean * mean
    hn = (h1 - mean) * jax.lax.rsqrt(var + 1e-5) * g_ref[...] + bt_ref[...]
    h2 = jnp.maximum(hn, 0.0)
    h3 = jnp.dot(h2, w2_ref[...], preferred_element_type=jnp.float32) \
        + b2_ref[...]
    r = jnp.maximum(h3, 0.0)
    o_ref[...] = r if first else x_ref[...] + r


def _row_block(i):
    return (i, 0)


def _dense1_call(eps, x, agg, w1, b1):
    return pl.pallas_call(
        _dense1_body,
        grid=(NB,),
        in_specs=[
            pl.BlockSpec(memory_space=pltpu.SMEM),
            pl.BlockSpec((RB, D), _row_block),
            pl.BlockSpec((2, RB, D), lambda i: (0, i, 0)),
            pl.BlockSpec((D, D), lambda i: (0, 0)),
            pl.BlockSpec((1, D), lambda i: (0, 0)),
        ],
        out_specs=[
            pl.BlockSpec((RB, D), _row_block),
            pl.BlockSpec((1, 1, D), lambda i: (i, 0, 0)),
            pl.BlockSpec((1, 1, D), lambda i: (i, 0, 0)),
        ],
        out_shape=[
            jax.ShapeDtypeStruct((N, D), jnp.float32),
            jax.ShapeDtypeStruct((NB, 1, D), jnp.float32),
            jax.ShapeDtypeStruct((NB, 1, D), jnp.float32),
        ],
    )(eps, x, agg, w1, b1)


def _dense2_call(first, x, h1, sums, sqs, gamma, beta, w2, b2):
    return pl.pallas_call(
        functools.partial(_dense2_body, first),
        grid=(NB,),
        in_specs=[
            pl.BlockSpec((RB, D), _row_block),
            pl.BlockSpec((RB, D), _row_block),
            pl.BlockSpec((NB, 1, D), lambda i: (0, 0, 0)),
            pl.BlockSpec((NB, 1, D), lambda i: (0, 0, 0)),
            pl.BlockSpec((1, D), lambda i: (0, 0)),
            pl.BlockSpec((1, D), lambda i: (0, 0)),
            pl.BlockSpec((D, D), lambda i: (0, 0)),
            pl.BlockSpec((1, D), lambda i: (0, 0)),
        ],
        out_specs=pl.BlockSpec((RB, D), _row_block),
        out_shape=jax.ShapeDtypeStruct((N, D), jnp.float32),
    )(x, h1, sums, sqs, gamma, beta, w2, b2)


def kernel(x, edge_index, edge_attr, params):
    x = x.astype(jnp.float32)
    src = edge_index[0].astype(jnp.int32)
    dst = edge_index[1].astype(jnp.int32)
    a = edge_attr.astype(jnp.float32).reshape(E)
    for li, p in enumerate(params):
        web = jnp.stack([p["We"].reshape(D), p["be"].reshape(D)])
        agg = _edge_call()(x, src, dst, a, web)
        eps = p["eps"].reshape(1)
        h1, sums, sqs = _dense1_call(eps, x, agg, p["W1"],
                                     p["b1"].reshape(1, D))
        x = _dense2_call(li == 0, x, h1, sums, sqs,
                         p["gamma"].reshape(1, D), p["beta"].reshape(1, D),
                         p["W2"], p["b2"].reshape(1, D))
    return x
